# baseline (speedup 1.0000x reference)
import functools

import jax
import jax.numpy as jnp
from jax import lax
from jax.experimental import pallas as pl
from jax.experimental.pallas import tpu as pltpu
from jax.experimental.pallas import tpu_sc as plsc

L = 16
NTILES = 32
CONF = 16384
ATOMS = 200
ROWS_PER_TILE = CONF // NTILES
GROUPS = ROWS_PER_TILE // L
QSTEPS = ATOMS // 4


def _sae_body(spec_hbm, en_hbm, se_hbm, out_hbm, spec_v, t8_v, t4_v, en_v, out_v):
    c = lax.axis_index("c")
    s = lax.axis_index("s")
    wid = s * 2 + c
    base = wid * ROWS_PER_TILE

    pltpu.sync_copy(se_hbm, t8_v)
    pltpu.sync_copy(en_hbm.at[pl.ds(base, ROWS_PER_TILE)], en_v)
    pltpu.sync_copy(
        spec_hbm.at[pl.ds(base * ATOMS, ROWS_PER_TILE * ATOMS)], spec_v)

    iota = lax.iota(jnp.int32, L)

    def build(v, carry):
        idx = iota + v * L
        a = (idx >> 9) & 7
        b = (idx >> 6) & 7
        cc = (idx >> 3) & 7
        d = idx & 7
        val = (plsc.load_gather(t8_v, [a]) + plsc.load_gather(t8_v, [b])
               + plsc.load_gather(t8_v, [cc]) + plsc.load_gather(t8_v, [d]))
        t4_v[pl.ds(v * L, L)] = val
        return carry

    lax.fori_loop(0, 4096 // L, build, 0)

    def group(g, carry):
        rowoff = (iota + g * L) * ATOMS
        acc0 = en_v[pl.ds(g * L, L)]

        def step(j, acc):
            cb = j * 4
            s0 = plsc.load_gather(spec_v, [rowoff + cb])
            s1 = plsc.load_gather(spec_v, [rowoff + (cb + 1)])
            s2 = plsc.load_gather(spec_v, [rowoff + (cb + 2)])
            s3 = plsc.load_gather(spec_v, [rowoff + (cb + 3)])
            pidx = (s0 << 9) | (s1 << 6) | (s2 << 3) | s3
            return acc + plsc.load_gather(t4_v, [pidx])

        out_v[pl.ds(g * L, L)] = lax.fori_loop(0, QSTEPS, step, acc0)
        return carry

    lax.fori_loop(0, GROUPS, group, 0)

    pltpu.sync_copy(out_v, out_hbm.at[pl.ds(base, ROWS_PER_TILE)])


def _make_sae():
    mesh = plsc.VectorSubcoreMesh(core_axis_name="c", subcore_axis_name="s")
    return functools.partial(
        pl.kernel,
        mesh=mesh,
        compiler_params=pltpu.CompilerParams(needs_layout_passes=False),
        out_type=jax.ShapeDtypeStruct((CONF,), jnp.float32),
        scratch_types=[
            pltpu.VMEM((ROWS_PER_TILE * ATOMS,), jnp.int32),
            pltpu.VMEM((8,), jnp.float32),
            pltpu.VMEM((4096,), jnp.float32),
            pltpu.VMEM((ROWS_PER_TILE,), jnp.float32),
            pltpu.VMEM((ROWS_PER_TILE,), jnp.float32),
        ],
    )(_sae_body)


def kernel(species, energies, self_energies):
    spec_flat = species.astype(jnp.int32).reshape(CONF * ATOMS)
    se8 = jnp.zeros((8,), jnp.float32).at[:7].set(self_energies.astype(jnp.float32))
    out = _make_sae()(spec_flat, energies.astype(jnp.float32), se8)
    return (species, out)
